# Initial kernel scaffold; baseline (speedup 1.0000x reference)
import jax
import jax.numpy as jnp
from jax.experimental import pallas as pl


def kernel(x, edge_index, edge_attr, emb, W_msg1, b_msg1, W_msg2, b_msg2, W_up1, b_up1, W_up2, b_up2):
    raise NotImplementedError("write your pallas kernel here")


# speedup vs baseline: 1.0542x; 1.0542x over previous
import functools

import jax
import jax.numpy as jnp
from jax import lax
from jax.experimental import pallas as pl
from jax.experimental.pallas import tpu as pltpu

N = 10000
E = 320000
D = 128
DE = 4
NP = 10240
BN = 1024
BE = 8000


def _prep_body(x_ref, t_ref, hw_ref, hu_ref):
    ids = x_ref[...]
    oh = (ids == lax.broadcasted_iota(jnp.int32, (BN, D), 1)).astype(jnp.float32)
    r = jnp.dot(oh, t_ref[...], preferred_element_type=jnp.float32)
    hw_ref[...] = r[:, :D]
    hu_ref[...] = r[:, D:]


def _edge_body(ea_ref, w_ref, b_ref, ew1_ref, ew2_ref):
    r = jnp.dot(ea_ref[...], w_ref[...], preferred_element_type=jnp.float32)
    r = r + b_ref[...]
    ew1_ref[...] = r[:, :D]
    ew2_ref[...] = r[:, D:]


def _up1_body(hu_ref, agg_ref, wu_ref, b_ref, w2_ref, hw2_ref, hu2_ref):
    h1 = jnp.maximum(
        hu_ref[...]
        + jnp.dot(agg_ref[...], wu_ref[...], preferred_element_type=jnp.float32)
        + b_ref[...], 0.0)
    r = jnp.dot(h1, w2_ref[...], preferred_element_type=jnp.float32)
    hw2_ref[...] = r[:, :D]
    hu2_ref[...] = r[:, D:]


def _up2_body(hu2_ref, agg_ref, wu_ref, b_ref, out_ref):
    i = pl.program_id(0)
    h2 = jnp.maximum(
        hu2_ref[...]
        + jnp.dot(agg_ref[...], wu_ref[...], preferred_element_type=jnp.float32)
        + b_ref[...], 0.0)
    rid = i * BN + lax.broadcasted_iota(jnp.int32, (BN, 1), 0)
    h2 = jnp.where(rid < N, h2, 0.0)

    @pl.when(i == 0)
    def _():
        out_ref[...] = jnp.zeros_like(out_ref)

    out_ref[...] += jnp.sum(h2, axis=0, keepdims=True)


def _prep(x_pad, t):
    return pl.pallas_call(
        _prep_body,
        grid=(NP // BN,),
        in_specs=[
            pl.BlockSpec((BN, 1), lambda i: (i, 0)),
            pl.BlockSpec((D, 2 * D), lambda i: (0, 0)),
        ],
        out_specs=[
            pl.BlockSpec((BN, D), lambda i: (i, 0)),
            pl.BlockSpec((BN, D), lambda i: (i, 0)),
        ],
        out_shape=[
            jax.ShapeDtypeStruct((NP, D), jnp.float32),
            jax.ShapeDtypeStruct((NP, D), jnp.float32),
        ],
    )(x_pad, t)


def _edge_pre(edge_attr, w, b):
    return pl.pallas_call(
        _edge_body,
        grid=(E // BE,),
        in_specs=[
            pl.BlockSpec((BE, DE), lambda i: (i, 0)),
            pl.BlockSpec((DE, 2 * D), lambda i: (0, 0)),
            pl.BlockSpec((1, 2 * D), lambda i: (0, 0)),
        ],
        out_specs=[
            pl.BlockSpec((BE, D), lambda i: (i, 0)),
            pl.BlockSpec((BE, D), lambda i: (i, 0)),
        ],
        out_shape=[
            jax.ShapeDtypeStruct((E, D), jnp.float32),
            jax.ShapeDtypeStruct((E, D), jnp.float32),
        ],
    )(edge_attr, w, b)


def _up1(hu1, agg1, wu1t, b_up1, w2):
    return pl.pallas_call(
        _up1_body,
        grid=(NP // BN,),
        in_specs=[
            pl.BlockSpec((BN, D), lambda i: (i, 0)),
            pl.BlockSpec((BN, D), lambda i: (i, 0)),
            pl.BlockSpec((D, D), lambda i: (0, 0)),
            pl.BlockSpec((1, D), lambda i: (0, 0)),
            pl.BlockSpec((D, 2 * D), lambda i: (0, 0)),
        ],
        out_specs=[
            pl.BlockSpec((BN, D), lambda i: (i, 0)),
            pl.BlockSpec((BN, D), lambda i: (i, 0)),
        ],
        out_shape=[
            jax.ShapeDtypeStruct((NP, D), jnp.float32),
            jax.ShapeDtypeStruct((NP, D), jnp.float32),
        ],
    )(hu1, agg1, wu1t, b_up1, w2)


def _up2(hu2, agg2, wu2t, b_up2):
    return pl.pallas_call(
        _up2_body,
        grid=(NP // BN,),
        in_specs=[
            pl.BlockSpec((BN, D), lambda i: (i, 0)),
            pl.BlockSpec((BN, D), lambda i: (i, 0)),
            pl.BlockSpec((D, D), lambda i: (0, 0)),
            pl.BlockSpec((1, D), lambda i: (0, 0)),
        ],
        out_specs=pl.BlockSpec((1, D), lambda i: (0, 0)),
        out_shape=jax.ShapeDtypeStruct((1, D), jnp.float32),
    )(hu2, agg2, wu2t, b_up2)


def _edge_round(hw, ew, src, dst):
    m = jnp.maximum(jnp.take(hw, src, axis=0) + ew, 0.0)
    return jnp.zeros((NP, D), jnp.float32).at[dst].add(m)


def kernel(x, edge_index, edge_attr, emb, W_msg1, b_msg1, W_msg2, b_msg2,
           W_up1, b_up1, W_up2, b_up2):
    src = edge_index[0]
    dst = edge_index[1]
    x_pad = jnp.pad(x, (0, NP - N)).reshape(NP, 1)

    t1 = emb @ W_msg1[:, :D].T
    tu1 = emb @ W_up1.T
    t = jnp.concatenate([t1, tu1], axis=1)
    we = jnp.concatenate([W_msg1[:, D:].T, W_msg2[:, D:].T], axis=1)
    be = jnp.concatenate([b_msg1, b_msg2]).reshape(1, 2 * D)
    w2 = jnp.concatenate([W_msg2[:, :D].T, W_up2.T], axis=1)

    hw1, hu1 = _prep(x_pad, t)
    ew1, ew2 = _edge_pre(edge_attr, we, be)

    agg1 = _edge_round(hw1, ew1, src, dst)
    hw2, hu2 = _up1(hu1, agg1, W_up1.T, b_up1.reshape(1, D), w2)
    agg2 = _edge_round(hw2, ew2, src, dst)
    out = _up2(hu2, agg2, W_up2.T, b_up2.reshape(1, D))
    return out[0] / N


# speedup vs baseline: 3.2151x; 3.0498x over previous
import functools

import jax
import jax.numpy as jnp
from jax import lax
from jax.experimental import pallas as pl
from jax.experimental.pallas import tpu as pltpu
from jax.experimental.pallas import tpu_sc as plsc

N = 10000
E = 320000
D = 128
DE = 4
NP = 10240
BN = 1024
BE = 8000

NTILES = 32
EPT = E // NTILES
CH = 128
NFULL = EPT // CH
TAIL = EPT - NFULL * CH
RPT = NP // 16


def _prep_body(x_ref, t_ref, hw_ref, hu_ref):
    ids = x_ref[...]
    oh = (ids == lax.broadcasted_iota(jnp.int32, (BN, D), 1)).astype(jnp.float32)
    r = jnp.dot(oh, t_ref[...], preferred_element_type=jnp.float32)
    hw_ref[...] = r[:, :D]
    hu_ref[...] = r[:, D:]


def _edge_body(ea_ref, w_ref, b_ref, ew1_ref, ew2_ref):
    r = jnp.dot(ea_ref[...], w_ref[...], preferred_element_type=jnp.float32)
    r = r + b_ref[...]
    ew1_ref[...] = r[:, :D]
    ew2_ref[...] = r[:, D:]


def _up1_body(hu_ref, agg_ref, wu_ref, b_ref, w2_ref, hw2_ref, hu2_ref):
    agg = agg_ref[0] + agg_ref[1]
    h1 = jnp.maximum(
        hu_ref[...]
        + jnp.dot(agg, wu_ref[...], preferred_element_type=jnp.float32)
        + b_ref[...], 0.0)
    r = jnp.dot(h1, w2_ref[...], preferred_element_type=jnp.float32)
    hw2_ref[...] = r[:, :D]
    hu2_ref[...] = r[:, D:]


def _up2_body(hu2_ref, agg_ref, wu_ref, b_ref, out_ref):
    i = pl.program_id(0)
    agg = agg_ref[0] + agg_ref[1]
    h2 = jnp.maximum(
        hu2_ref[...]
        + jnp.dot(agg, wu_ref[...], preferred_element_type=jnp.float32)
        + b_ref[...], 0.0)
    rid = i * BN + lax.broadcasted_iota(jnp.int32, (BN, 1), 0)
    h2 = jnp.where(rid < N, h2, 0.0)

    @pl.when(i == 0)
    def _():
        out_ref[...] = jnp.zeros_like(out_ref)

    out_ref[...] += jnp.sum(h2, axis=0, keepdims=True)


def _prep(x_pad, t):
    return pl.pallas_call(
        _prep_body,
        grid=(NP // BN,),
        in_specs=[
            pl.BlockSpec((BN, 1), lambda i: (i, 0)),
            pl.BlockSpec((D, 2 * D), lambda i: (0, 0)),
        ],
        out_specs=[
            pl.BlockSpec((BN, D), lambda i: (i, 0)),
            pl.BlockSpec((BN, D), lambda i: (i, 0)),
        ],
        out_shape=[
            jax.ShapeDtypeStruct((NP, D), jnp.float32),
            jax.ShapeDtypeStruct((NP, D), jnp.float32),
        ],
    )(x_pad, t)


def _edge_pre(edge_attr, w, b):
    return pl.pallas_call(
        _edge_body,
        grid=(E // BE,),
        in_specs=[
            pl.BlockSpec((BE, DE), lambda i: (i, 0)),
            pl.BlockSpec((DE, 2 * D), lambda i: (0, 0)),
            pl.BlockSpec((1, 2 * D), lambda i: (0, 0)),
        ],
        out_specs=[
            pl.BlockSpec((BE, D), lambda i: (i, 0)),
            pl.BlockSpec((BE, D), lambda i: (i, 0)),
        ],
        out_shape=[
            jax.ShapeDtypeStruct((E, D), jnp.float32),
            jax.ShapeDtypeStruct((E, D), jnp.float32),
        ],
    )(edge_attr, w, b)


def _up1(hu1, agg1, wu1t, b_up1, w2):
    return pl.pallas_call(
        _up1_body,
        grid=(NP // BN,),
        in_specs=[
            pl.BlockSpec((BN, D), lambda i: (i, 0)),
            pl.BlockSpec((2, BN, D), lambda i: (0, i, 0)),
            pl.BlockSpec((D, D), lambda i: (0, 0)),
            pl.BlockSpec((1, D), lambda i: (0, 0)),
            pl.BlockSpec((D, 2 * D), lambda i: (0, 0)),
        ],
        out_specs=[
            pl.BlockSpec((BN, D), lambda i: (i, 0)),
            pl.BlockSpec((BN, D), lambda i: (i, 0)),
        ],
        out_shape=[
            jax.ShapeDtypeStruct((NP, D), jnp.float32),
            jax.ShapeDtypeStruct((NP, D), jnp.float32),
        ],
    )(hu1, agg1, wu1t, b_up1, w2)


def _up2(hu2, agg2, wu2t, b_up2):
    return pl.pallas_call(
        _up2_body,
        grid=(NP // BN,),
        in_specs=[
            pl.BlockSpec((BN, D), lambda i: (i, 0)),
            pl.BlockSpec((2, BN, D), lambda i: (0, i, 0)),
            pl.BlockSpec((D, D), lambda i: (0, 0)),
            pl.BlockSpec((1, D), lambda i: (0, 0)),
        ],
        out_specs=pl.BlockSpec((1, D), lambda i: (0, 0)),
        out_shape=jax.ShapeDtypeStruct((1, D), jnp.float32),
    )(hu2, agg2, wu2t, b_up2)


def _sc_agg(hw, ew, src, dst):
    mesh = plsc.VectorSubcoreMesh(core_axis_name="c", subcore_axis_name="s")

    @functools.partial(
        pl.kernel,
        mesh=mesh,
        out_type=jax.ShapeDtypeStruct((2, NP, D), jnp.float32),
        scratch_types=[
            pltpu.VMEM((CH,), jnp.int32),
            pltpu.VMEM((CH,), jnp.int32),
            pltpu.VMEM((CH, D), jnp.float32),
            pltpu.VMEM((CH, D), jnp.float32),
            pltpu.VMEM((TAIL,), jnp.int32),
            pltpu.VMEM((TAIL,), jnp.int32),
            pltpu.VMEM((TAIL, D), jnp.float32),
            pltpu.VMEM((TAIL, D), jnp.float32),
            pltpu.VMEM_SHARED((NP, D), jnp.float32),
            pltpu.SemaphoreType.DMA,
        ],
    )
    def k(hw_hbm, ew_hbm, src_hbm, dst_hbm, out_hbm,
          si, di, mb, gb, si_t, di_t, mb_t, gb_t, agg_sh, sem):
        c = lax.axis_index("c")
        s = lax.axis_index("s")

        def zrow(e, carry):
            for j in range(D // 16):
                mb[e, pl.ds(16 * j, 16)] = jnp.zeros((16,), jnp.float32)
            return carry
        lax.fori_loop(0, CH, zrow, 0)
        for t in range(RPT // CH):
            pltpu.sync_copy(mb, agg_sh.at[pl.ds(s * RPT + t * CH, CH)])
        plsc.subcore_barrier()

        base = (c * 16 + s) * EPT

        def do_chunk(off, n, si, di, mb, gb):
            pltpu.sync_copy(src_hbm.at[pl.ds(off, n)], si)
            pltpu.sync_copy(dst_hbm.at[pl.ds(off, n)], di)
            pltpu.async_copy(hw_hbm.at[si], gb, sem).wait()
            pltpu.sync_copy(ew_hbm.at[pl.ds(off, n)], mb)

            def crow(e, carry):
                for j in range(D // 16):
                    sl = pl.ds(16 * j, 16)
                    mb[e, sl] = jnp.maximum(mb[e, sl] + gb[e, sl], 0.0)
                return carry
            lax.fori_loop(0, n, crow, 0)
            pltpu.sync_copy(mb, agg_sh.at[di], add=True)

        def chunk_body(i, carry):
            do_chunk(base + i * CH, CH, si, di, mb, gb)
            return carry
        lax.fori_loop(0, NFULL, chunk_body, 0)
        do_chunk(base + NFULL * CH, TAIL, si_t, di_t, mb_t, gb_t)

        plsc.subcore_barrier()
        for t in range(RPT // CH):
            r0 = s * RPT + t * CH
            pltpu.sync_copy(agg_sh.at[pl.ds(r0, CH)],
                            out_hbm.at[c, pl.ds(r0, CH)])

    return k(hw, ew, src, dst)


def kernel(x, edge_index, edge_attr, emb, W_msg1, b_msg1, W_msg2, b_msg2,
           W_up1, b_up1, W_up2, b_up2):
    src = edge_index[0]
    dst = edge_index[1]
    x_pad = jnp.pad(x, (0, NP - N)).reshape(NP, 1)

    t1 = emb @ W_msg1[:, :D].T
    tu1 = emb @ W_up1.T
    t = jnp.concatenate([t1, tu1], axis=1)
    we = jnp.concatenate([W_msg1[:, D:].T, W_msg2[:, D:].T], axis=1)
    be = jnp.concatenate([b_msg1, b_msg2]).reshape(1, 2 * D)
    w2 = jnp.concatenate([W_msg2[:, :D].T, W_up2.T], axis=1)

    hw1, hu1 = _prep(x_pad, t)
    ew1, ew2 = _edge_pre(edge_attr, we, be)

    agg1 = _sc_agg(hw1, ew1, src, dst)
    hw2, hu2 = _up1(hu1, agg1, W_up1.T, b_up1.reshape(1, D), w2)
    agg2 = _sc_agg(hw2, ew2, src, dst)
    out = _up2(hu2, agg2, W_up2.T, b_up2.reshape(1, D))
    return out[0] / N
